# baseline (speedup 1.0000x reference)
import functools
import jax
import jax.numpy as jnp
from jax import lax
from jax.experimental import pallas as pl
from jax.experimental.pallas import tpu as pltpu
from jax.experimental.pallas import tpu_sc as plsc


def kernel(x, row_embed):
    B = x.shape[0]
    W, D = row_embed.shape
    NC, NS = 2, 16
    NW = NC * NS
    bpw = B // NW
    RB = 8
    P = 4
    PW = W // P
    NCH = bpw // RB
    F = 5

    rep4 = jnp.broadcast_to(row_embed.reshape(P, PW, 1, D), (P, PW, RB, D))
    rep2 = rep4.reshape(2, 2 * PW, RB, D)

    mesh = plsc.VectorSubcoreMesh(
        core_axis_name="c", subcore_axis_name="s", num_cores=NC, num_subcores=NS
    )

    @functools.partial(
        pl.kernel,
        mesh=mesh,
        out_type=jax.ShapeDtypeStruct((W, B, D), jnp.float32),
        scratch_types=[
            pltpu.VMEM((2, PW, RB, D), jnp.float32),
            pltpu.VMEM_SHARED((2 * PW, RB, D), jnp.float32),
            pltpu.SemaphoreType.DMA,
            pltpu.SemaphoreType.DMA,
            pltpu.SemaphoreType.DMA,
        ],
    )
    def sc_broadcast(rep4_hbm, rep2_hbm, out_hbm, rep_v, shr_v, sem_f, sem_t, sem_s):
        c = lax.axis_index("c")
        s = lax.axis_index("s")
        wid = s * NC + c
        base = wid * bpw

        fetches = {0: pltpu.async_copy(rep4_hbm.at[0], rep_v.at[0], sem_f)}
        tile_outs = {}
        shr_outs = []
        for p in range(P):
            b = p % 2
            if p in (0, 2):
                for cp in shr_outs:
                    cp.wait()
                shr_outs = []
                plsc.subcore_barrier()

                @pl.when(s == 0)
                def _stage_shared():
                    pltpu.sync_copy(rep2_hbm.at[p // 2], shr_v)

                plsc.subcore_barrier()
            fetches[p].wait()
            tile_outs[p] = []
            for i in range(NCH):
                dst = out_hbm.at[pl.ds(p * PW, PW), pl.ds(base + i * RB, RB)]
                if i < F:
                    shr_outs.append(
                        pltpu.async_copy(shr_v.at[pl.ds(b * PW, PW)], dst, sem_s)
                    )
                else:
                    tile_outs[p].append(pltpu.async_copy(rep_v.at[b], dst, sem_t))
            if p + 1 < P:
                if p >= 1:
                    for cp in tile_outs[p - 1]:
                        cp.wait()
                fetches[p + 1] = pltpu.async_copy(
                    rep4_hbm.at[p + 1], rep_v.at[(p + 1) % 2], sem_f
                )
        for p in (P - 2, P - 1):
            for cp in tile_outs[p]:
                cp.wait()
        for cp in shr_outs:
            cp.wait()

    return jnp.transpose(sc_broadcast(rep4, rep2), (1, 0, 2))
